# baseline (speedup 1.0000x reference)
import jax
import jax.numpy as jnp
from jax.experimental import pallas as pl
from jax.experimental.pallas import tpu as pltpu

_BI = 32
_G = 4


def _blockdiag(m, g):
    z = jnp.zeros_like(m)
    out_rows = []
    for r in range(g):
        out_rows.append(jnp.concatenate(
            [m if c == r else z for c in range(g)], axis=1))
    return jnp.concatenate(out_rows, axis=0)


def _fused(x_ref, w0_ref, b0_ref, w1_ref, b1_ref, w2t_ref, b2_ref,
           fw0_ref, fb0_ref, fw1_ref, fb1_ref,
           gw0_ref, gb0_ref, gw1_ref, gb1_ref, gw2_ref, gb2_ref,
           out_ref, a_ref, pd_ref):
    x = x_ref[:, :]
    n = x.shape[0]
    h = n // _G
    fdim = x.shape[1]
    w0 = w0_ref[:, :]
    p = x @ w0[:fdim, :] + b0_ref[:, :]
    pd_ref[:, :] = jnp.concatenate([p] * _G, axis=1)
    q = x @ w0[fdim:, :]
    q2 = jnp.concatenate(
        [q[c * h:(c + 1) * h, :] for c in range(_G)], axis=1)

    w1dd = _blockdiag(w1_ref[:, :], _G).astype(jnp.bfloat16)
    b1 = b1_ref[:, :]
    b1d = jnp.concatenate([b1] * _G, axis=1)
    w2t = w2t_ref[:, :]
    w2dd = _blockdiag(w2t.T, _G).astype(jnp.bfloat16)
    b2 = b2_ref[0, 0]

    def body(i, carry):
        pi = pd_ref[pl.ds(i * _BI, _BI), :]
        t0 = jnp.maximum(pi[:, None, :] + q2[None, :, :], 0.0)
        t0b = t0.astype(jnp.bfloat16)
        h1 = jnp.maximum(
            jax.lax.dot(t0b.reshape(_BI * h, 64 * _G), w1dd,
                        preferred_element_type=jnp.float32) + b1d,
            0.0)
        ep = jax.lax.dot(h1.astype(jnp.bfloat16), w2dd,
                         preferred_element_type=jnp.float32)
        et = jnp.swapaxes(ep.reshape(_BI, h, _G), 1, 2)
        e = jnp.concatenate([et[:, c, :] for c in range(_G)], axis=-1)
        a_ref[pl.ds(i * _BI, _BI), :] = jax.nn.sigmoid(e + b2)
        return carry

    jax.lax.fori_loop(0, n // _BI, body, 0)

    a = a_ref[:, :]
    f = jnp.maximum(x @ fw0_ref[:, :] + fb0_ref[:, :], 0.0)
    f = f @ fw1_ref[:, :] + fb1_ref[:, :]
    g = jnp.maximum(a @ (f @ gw0_ref[:, :]) + gb0_ref[:, :], 0.0)
    g = jnp.maximum(a @ (g @ gw1_ref[:, :]) + gb1_ref[:, :], 0.0)
    out64 = a @ (g @ gw2_ref[:, :]) + gb2_ref[:, :]
    out_ref[:, :] = jnp.mean(out64, axis=1, keepdims=True)


def kernel(features, ec_W0, ec_b0, ec_W1, ec_b1, ec_W2, ec_b2,
           fc_W0, fc_b0, fc_W1, fc_b1,
           gcn_W0, gcn_b0, gcn_W1, gcn_b1, gcn_W2, gcn_b2):
    x = jnp.squeeze(features)
    n = x.shape[0]
    out = pl.pallas_call(
        _fused,
        out_shape=jax.ShapeDtypeStruct((n, 1), jnp.float32),
        scratch_shapes=[pltpu.VMEM((n, n), jnp.float32),
                        pltpu.VMEM((n, 64 * _G), jnp.float32)],
    )(x, ec_W0, ec_b0.reshape(1, -1), ec_W1, ec_b1.reshape(1, -1),
      ec_W2.reshape(1, -1), ec_b2.reshape(1, 1),
      fc_W0, fc_b0.reshape(1, -1), fc_W1, fc_b1.reshape(1, -1),
      gcn_W0, gcn_b0.reshape(1, -1), gcn_W1, gcn_b1.reshape(1, -1),
      gcn_W2, gcn_b2.reshape(1, -1))
    return out.reshape(n)


# speedup vs baseline: 1.0760x; 1.0760x over previous
import jax
import jax.numpy as jnp
from jax.experimental import pallas as pl
from jax.experimental.pallas import tpu as pltpu

_BI = 64
_G = 4


def _blockdiag(m, g):
    z = jnp.zeros_like(m)
    out_rows = []
    for r in range(g):
        out_rows.append(jnp.concatenate(
            [m if c == r else z for c in range(g)], axis=1))
    return jnp.concatenate(out_rows, axis=0)


def _fused(x_ref, w0_ref, b0_ref, w1_ref, b1_ref, w2t_ref, b2_ref,
           fw0_ref, fb0_ref, fw1_ref, fb1_ref,
           gw0_ref, gb0_ref, gw1_ref, gb1_ref, gw2_ref, gb2_ref,
           out_ref, a_ref, pd_ref):
    x = x_ref[:, :]
    n = x.shape[0]
    h = n // _G
    fdim = x.shape[1]
    w0 = w0_ref[:, :]
    p = x @ w0[:fdim, :] + b0_ref[:, :]
    pd_ref[:, :] = jnp.concatenate([p] * _G, axis=1)
    q = x @ w0[fdim:, :]
    q2 = jnp.concatenate(
        [q[c * h:(c + 1) * h, :] for c in range(_G)], axis=1)

    w1dd = _blockdiag(w1_ref[:, :], _G).astype(jnp.bfloat16)
    b1 = b1_ref[:, :]
    b1d = jnp.concatenate([b1] * _G, axis=1)
    w2t = w2t_ref[:, :]
    w2dd = _blockdiag(w2t.T, _G).astype(jnp.bfloat16)
    b2 = b2_ref[0, 0]

    def body(i, carry):
        pi = pd_ref[pl.ds(i * _BI, _BI), :]
        t0 = jnp.maximum(pi[:, None, :] + q2[None, :, :], 0.0)
        t0b = t0.astype(jnp.bfloat16)
        h1 = jnp.maximum(
            jax.lax.dot(t0b.reshape(_BI * h, 64 * _G), w1dd,
                        preferred_element_type=jnp.float32) + b1d,
            0.0)
        ep = jax.lax.dot(h1.astype(jnp.bfloat16), w2dd,
                         preferred_element_type=jnp.float32)
        et = jnp.swapaxes(ep.reshape(_BI, h, _G), 1, 2)
        e = jnp.concatenate([et[:, c, :] for c in range(_G)], axis=-1)
        a_ref[pl.ds(i * _BI, _BI), :] = jax.nn.sigmoid(e + b2)
        return carry

    jax.lax.fori_loop(0, n // _BI, body, 0)

    a = a_ref[:, :]
    f = jnp.maximum(x @ fw0_ref[:, :] + fb0_ref[:, :], 0.0)
    f = f @ fw1_ref[:, :] + fb1_ref[:, :]
    g = jnp.maximum(a @ (f @ gw0_ref[:, :]) + gb0_ref[:, :], 0.0)
    g = jnp.maximum(a @ (g @ gw1_ref[:, :]) + gb1_ref[:, :], 0.0)
    out64 = a @ (g @ gw2_ref[:, :]) + gb2_ref[:, :]
    out_ref[:, :] = jnp.mean(out64, axis=1, keepdims=True)


def kernel(features, ec_W0, ec_b0, ec_W1, ec_b1, ec_W2, ec_b2,
           fc_W0, fc_b0, fc_W1, fc_b1,
           gcn_W0, gcn_b0, gcn_W1, gcn_b1, gcn_W2, gcn_b2):
    x = jnp.squeeze(features)
    n = x.shape[0]
    out = pl.pallas_call(
        _fused,
        out_shape=jax.ShapeDtypeStruct((n, 1), jnp.float32),
        scratch_shapes=[pltpu.VMEM((n, n), jnp.float32),
                        pltpu.VMEM((n, 64 * _G), jnp.float32)],
    )(x, ec_W0, ec_b0.reshape(1, -1), ec_W1, ec_b1.reshape(1, -1),
      ec_W2.reshape(1, -1), ec_b2.reshape(1, 1),
      fc_W0, fc_b0.reshape(1, -1), fc_W1, fc_b1.reshape(1, -1),
      gcn_W0, gcn_b0.reshape(1, -1), gcn_W1, gcn_b1.reshape(1, -1),
      gcn_W2, gcn_b2.reshape(1, -1))
    return out.reshape(n)


# speedup vs baseline: 1.1237x; 1.0443x over previous
import jax
import jax.numpy as jnp
from jax.experimental import pallas as pl
from jax.experimental.pallas import tpu as pltpu

_BI = 128
_G = 4


def _blockdiag(m, g):
    z = jnp.zeros_like(m)
    out_rows = []
    for r in range(g):
        out_rows.append(jnp.concatenate(
            [m if c == r else z for c in range(g)], axis=1))
    return jnp.concatenate(out_rows, axis=0)


def _fused(x_ref, w0_ref, b0_ref, w1_ref, b1_ref, w2t_ref, b2_ref,
           fw0_ref, fb0_ref, fw1_ref, fb1_ref,
           gw0_ref, gb0_ref, gw1_ref, gb1_ref, gw2_ref, gb2_ref,
           out_ref, a_ref, pd_ref):
    x = x_ref[:, :]
    n = x.shape[0]
    h = n // _G
    fdim = x.shape[1]
    w0 = w0_ref[:, :]
    p = x @ w0[:fdim, :] + b0_ref[:, :]
    pd_ref[:, :] = jnp.concatenate([p] * _G, axis=1)
    q = x @ w0[fdim:, :]
    q2 = jnp.concatenate(
        [q[c * h:(c + 1) * h, :] for c in range(_G)], axis=1)

    w1dd = _blockdiag(w1_ref[:, :], _G).astype(jnp.bfloat16)
    b1 = b1_ref[:, :]
    b1d = jnp.concatenate([b1] * _G, axis=1)
    w2t = w2t_ref[:, :]
    w2dd = _blockdiag(w2t.T, _G).astype(jnp.bfloat16)
    b2 = b2_ref[0, 0]

    def body(i, carry):
        pi = pd_ref[pl.ds(i * _BI, _BI), :]
        t0 = jnp.maximum(pi[:, None, :] + q2[None, :, :], 0.0)
        t0b = t0.astype(jnp.bfloat16)
        h1 = jnp.maximum(
            jax.lax.dot(t0b.reshape(_BI * h, 64 * _G), w1dd,
                        preferred_element_type=jnp.float32) + b1d,
            0.0)
        ep = jax.lax.dot(h1.astype(jnp.bfloat16), w2dd,
                         preferred_element_type=jnp.float32)
        et = jnp.swapaxes(ep.reshape(_BI, h, _G), 1, 2)
        e = jnp.concatenate([et[:, c, :] for c in range(_G)], axis=-1)
        a_ref[pl.ds(i * _BI, _BI), :] = jax.nn.sigmoid(e + b2)
        return carry

    jax.lax.fori_loop(0, n // _BI, body, 0)

    a = a_ref[:, :]
    f = jnp.maximum(x @ fw0_ref[:, :] + fb0_ref[:, :], 0.0)
    f = f @ fw1_ref[:, :] + fb1_ref[:, :]
    g = jnp.maximum(a @ (f @ gw0_ref[:, :]) + gb0_ref[:, :], 0.0)
    g = jnp.maximum(a @ (g @ gw1_ref[:, :]) + gb1_ref[:, :], 0.0)
    out64 = a @ (g @ gw2_ref[:, :]) + gb2_ref[:, :]
    out_ref[:, :] = jnp.mean(out64, axis=1, keepdims=True)


def kernel(features, ec_W0, ec_b0, ec_W1, ec_b1, ec_W2, ec_b2,
           fc_W0, fc_b0, fc_W1, fc_b1,
           gcn_W0, gcn_b0, gcn_W1, gcn_b1, gcn_W2, gcn_b2):
    x = jnp.squeeze(features)
    n = x.shape[0]
    out = pl.pallas_call(
        _fused,
        out_shape=jax.ShapeDtypeStruct((n, 1), jnp.float32),
        scratch_shapes=[pltpu.VMEM((n, n), jnp.float32),
                        pltpu.VMEM((n, 64 * _G), jnp.float32)],
    )(x, ec_W0, ec_b0.reshape(1, -1), ec_W1, ec_b1.reshape(1, -1),
      ec_W2.reshape(1, -1), ec_b2.reshape(1, 1),
      fc_W0, fc_b0.reshape(1, -1), fc_W1, fc_b1.reshape(1, -1),
      gcn_W0, gcn_b0.reshape(1, -1), gcn_W1, gcn_b1.reshape(1, -1),
      gcn_W2, gcn_b2.reshape(1, -1))
    return out.reshape(n)


# speedup vs baseline: 1.1244x; 1.0006x over previous
import jax
import jax.numpy as jnp
from jax.experimental import pallas as pl
from jax.experimental.pallas import tpu as pltpu

_BI = 128
_G = 4


def _blockdiag(m, g):
    z = jnp.zeros_like(m)
    out_rows = []
    for r in range(g):
        out_rows.append(jnp.concatenate(
            [m if c == r else z for c in range(g)], axis=1))
    return jnp.concatenate(out_rows, axis=0)


def _fused(x_ref, w0_ref, b0_ref, w1_ref, b1_ref, w2t_ref, b2_ref,
           fw0_ref, fb0_ref, fw1_ref, fb1_ref,
           gw0_ref, gb0_ref, gw1_ref, gb1_ref, gw2_ref, gb2_ref,
           out_ref, a_ref, pd_ref):
    x = x_ref[:, :]
    n = x.shape[0]
    h = n // _G
    fdim = x.shape[1]
    w0 = w0_ref[:, :]
    p = x @ w0[:fdim, :] + b0_ref[:, :]
    pd_ref[:, :] = jnp.concatenate([p] * _G, axis=1)
    q = x @ w0[fdim:, :]
    q2 = jnp.concatenate(
        [q[c * h:(c + 1) * h, :] for c in range(_G)], axis=1)

    w1dd = _blockdiag(w1_ref[:, :], _G).astype(jnp.bfloat16)
    b1 = b1_ref[:, :]
    b1d = jnp.concatenate([b1] * _G, axis=1)
    w2t = w2t_ref[:, :]
    w2dd = _blockdiag(w2t.T, _G).astype(jnp.bfloat16)
    b2 = b2_ref[0, 0]

    def body(i, carry):
        pi = pd_ref[pl.ds(i * _BI, _BI), :]
        t0 = jnp.maximum(pi[:, None, :] + q2[None, :, :], 0.0)
        t0b = t0.astype(jnp.bfloat16)
        h1b = jnp.maximum(
            jax.lax.dot(t0b.reshape(_BI * h, 64 * _G), w1dd,
                        preferred_element_type=jnp.float32) + b1d,
            0.0).astype(jnp.bfloat16)
        ep = jax.lax.dot(h1b, w2dd, preferred_element_type=jnp.float32)
        et = jnp.swapaxes(ep.reshape(_BI, h, _G), 1, 2)
        e = jnp.concatenate([et[:, c, :] for c in range(_G)], axis=-1)
        a_ref[pl.ds(i * _BI, _BI), :] = jax.nn.sigmoid(e + b2)
        return carry

    jax.lax.fori_loop(0, n // _BI, body, 0)

    a = a_ref[:, :]
    f = jnp.maximum(x @ fw0_ref[:, :] + fb0_ref[:, :], 0.0)
    f = f @ fw1_ref[:, :] + fb1_ref[:, :]
    g = jnp.maximum(a @ (f @ gw0_ref[:, :]) + gb0_ref[:, :], 0.0)
    g = jnp.maximum(a @ (g @ gw1_ref[:, :]) + gb1_ref[:, :], 0.0)
    out64 = a @ (g @ gw2_ref[:, :]) + gb2_ref[:, :]
    out_ref[:, :] = jnp.mean(out64, axis=1, keepdims=True)


def kernel(features, ec_W0, ec_b0, ec_W1, ec_b1, ec_W2, ec_b2,
           fc_W0, fc_b0, fc_W1, fc_b1,
           gcn_W0, gcn_b0, gcn_W1, gcn_b1, gcn_W2, gcn_b2):
    x = jnp.squeeze(features)
    n = x.shape[0]
    out = pl.pallas_call(
        _fused,
        out_shape=jax.ShapeDtypeStruct((n, 1), jnp.float32),
        scratch_shapes=[pltpu.VMEM((n, n), jnp.float32),
                        pltpu.VMEM((n, 64 * _G), jnp.float32)],
    )(x, ec_W0, ec_b0.reshape(1, -1), ec_W1, ec_b1.reshape(1, -1),
      ec_W2.reshape(1, -1), ec_b2.reshape(1, 1),
      fc_W0, fc_b0.reshape(1, -1), fc_W1, fc_b1.reshape(1, -1),
      gcn_W0, gcn_b0.reshape(1, -1), gcn_W1, gcn_b1.reshape(1, -1),
      gcn_W2, gcn_b2.reshape(1, -1))
    return out.reshape(n)
